# Initial kernel scaffold; baseline (speedup 1.0000x reference)
import jax
import jax.numpy as jnp
from jax.experimental import pallas as pl


def kernel(x, edge_index, W_proj, b_proj, Wq1, Wk1, Wv1, Wq2, Wk2, Wv2, Ws, bs):
    raise NotImplementedError("write your pallas kernel here")


# speedup vs baseline: 1.6424x; 1.6424x over previous
import functools

import jax
import jax.numpy as jnp
from jax.experimental import pallas as pl
from jax.experimental.pallas import tpu as pltpu

N = 10000
E = 320000
F_IN = 128
H = 64


def _gelu(x):
    return 0.5 * x * (1.0 + jax.lax.erf(x * (2.0 ** -0.5)))


def _dense1_body(x_ref, wp_ref, bp_ref, wq_ref, wk_ref, wv_ref,
                 h_ref, q_ref, k_ref, v_ref):
    h = _gelu(jnp.dot(x_ref[...], wp_ref[...],
                      preferred_element_type=jnp.float32) + bp_ref[...])
    h_ref[...] = h
    q_ref[...] = jnp.dot(h, wq_ref[...], preferred_element_type=jnp.float32)
    k_ref[...] = jnp.dot(h, wk_ref[...], preferred_element_type=jnp.float32)
    v_ref[...] = jnp.dot(h, wv_ref[...], preferred_element_type=jnp.float32)


def _dense2_body(h_ref, wq_ref, wk_ref, wv_ref, q_ref, k_ref, v_ref):
    h = h_ref[...]
    q_ref[...] = jnp.dot(h, wq_ref[...], preferred_element_type=jnp.float32)
    k_ref[...] = jnp.dot(h, wk_ref[...], preferred_element_type=jnp.float32)
    v_ref[...] = jnp.dot(h, wv_ref[...], preferred_element_type=jnp.float32)


def _sig_body(h_ref, ws_ref, bs_ref, out_ref):
    out_ref[...] = jnp.dot(h_ref[...], ws_ref[...],
                           preferred_element_type=jnp.float32) + bs_ref[...]


def _edge_layer(q, k, v, src, dst):
    q_i = q[dst]
    k_j = k[src]
    alpha = jax.nn.relu((q_i * k_j).sum(axis=-1) / (H ** 0.5))
    ex = jnp.exp(alpha)
    denom = jax.ops.segment_sum(ex, dst, num_segments=N)
    a = ex / (denom[dst] + 1e-16)
    msg = v[src] * a[:, None]
    out = jax.ops.segment_sum(msg, dst, num_segments=N)
    return out, a


def kernel(x, edge_index, W_proj, b_proj, Wq1, Wk1, Wv1, Wq2, Wk2, Wv2, Ws, bs):
    src = edge_index[0]
    dst = edge_index[1]
    f32 = jnp.float32
    h, q1, k1, v1 = pl.pallas_call(
        _dense1_body,
        out_shape=[jax.ShapeDtypeStruct((N, H), f32)] * 4,
    )(x, W_proj, b_proj.reshape(1, H), Wq1, Wk1, Wv1)

    m1, a1 = _edge_layer(q1, k1, v1, src, dst)
    h1 = jax.nn.gelu(m1, approximate=False)

    q2, k2, v2 = pl.pallas_call(
        _dense2_body,
        out_shape=[jax.ShapeDtypeStruct((N, H), f32)] * 3,
    )(h1, Wq2, Wk2, Wv2)

    h2, a2 = _edge_layer(q2, k2, v2, src, dst)

    signals = pl.pallas_call(
        _sig_body,
        out_shape=jax.ShapeDtypeStruct((N, 1), f32),
    )(h2, Ws, bs.reshape(1, 1))

    return (signals, a1[:, None], a2[:, None])


# speedup vs baseline: 3.1849x; 1.9393x over previous
import functools

import jax
import jax.numpy as jnp
from jax import lax
from jax.experimental import pallas as pl
from jax.experimental.pallas import tpu as pltpu
from jax.experimental.pallas import tpu_sc as plsc

N = 10000
E = 320000
F_IN = 128
H = 64
VW = 128
MW = 80
NC = 2
NS = 16
NW = NC * NS
CE = 128
NCHG = E // CE
JMAX_E = (NCHG + NW - 1) // NW
GE = CE // 16
CN = 512
NCHN = E // CN
JMAX_N = (NCHN + NW - 1) // NW
GN = CN // 16
NP = 10240
RPT = NP // NS
ZR = 64
SB = 64
_SCALE = 1.0 / (H ** 0.5)
_EPS = 1e-16

f32 = jnp.float32
i32 = jnp.int32


def _gelu(x):
    return 0.5 * x * (1.0 + lax.erf(x * (2.0 ** -0.5)))


def _qkv_pad(h, wq_ref, wk_ref, wv_ref, q_ref, k_ref, v_ref):
    zero_pad = jnp.zeros((N, VW - H), f32)
    q_ref[:, :H] = jnp.dot(h, wq_ref[...], preferred_element_type=f32)
    q_ref[:, H:] = zero_pad
    k_ref[:, :H] = jnp.dot(h, wk_ref[...], preferred_element_type=f32)
    k_ref[:, H:] = zero_pad
    v_ref[:, :H] = jnp.dot(h, wv_ref[...], preferred_element_type=f32)
    colv = lax.broadcasted_iota(i32, (N, VW - H), 1)
    v_ref[:, H:] = jnp.where(colv == 0, 1.0, 0.0).astype(f32)


def _dense1_body(x_ref, wp_ref, bp_ref, wq_ref, wk_ref, wv_ref,
                 q_ref, k_ref, v_ref):
    h = _gelu(jnp.dot(x_ref[...], wp_ref[...],
                      preferred_element_type=f32) + bp_ref[...])
    _qkv_pad(h, wq_ref, wk_ref, wv_ref, q_ref, k_ref, v_ref)


def _mid_body(h1_ref, wq_ref, wk_ref, wv_ref, q_ref, k_ref, v_ref):
    _qkv_pad(h1_ref[...], wq_ref, wk_ref, wv_ref, q_ref, k_ref, v_ref)


def _fin_body(h2_ref, ws_ref, bs_ref, sig_ref):
    sig_ref[...] = jnp.dot(h2_ref[...], ws_ref[...],
                           preferred_element_type=f32) + bs_ref[...]


_sc_mesh = plsc.VectorSubcoreMesh(core_axis_name="c", subcore_axis_name="s")


@functools.partial(
    pl.kernel,
    out_type=[jax.ShapeDtypeStruct((NW, JMAX_E, 1, CE), f32),
              jax.ShapeDtypeStruct((2, NP, MW), f32)],
    mesh=_sc_mesh,
    scratch_types=[
        pltpu.VMEM((16,), i32),
        pltpu.VMEM((CE,), f32),
        pltpu.VMEM((16, CE), i32),
        pltpu.VMEM((16, CE), i32),
        pltpu.VMEM((SB,), i32),
        pltpu.VMEM((SB,), i32),
        pltpu.VMEM((SB, VW), f32),
        pltpu.VMEM((SB, VW), f32),
        pltpu.VMEM((SB, VW), f32),
        pltpu.VMEM((SB, MW), f32),
        pltpu.VMEM((ZR, MW), f32),
        pltpu.VMEM_SHARED((NP, MW), f32),
        pltpu.SemaphoreType.DMA,
        pltpu.SemaphoreType.DMA,
        pltpu.SemaphoreType.DMA,
    ],
    compiler_params=pltpu.CompilerParams(needs_layout_passes=False),
)
def _edge_pass(q_hbm, k_hbm, v_hbm, src2_hbm, dst2_hbm, ex_hbm, mp_hbm,
               cidxb, exb, srcb2, dstb2, srcA, dstA, qrows, krows, vrows,
               vmsg, zbuf, m_sh, s1, s2, s3):
    cid = lax.axis_index("c")
    sid = lax.axis_index("s")
    wid = cid * NS + sid

    z16 = jnp.zeros((16,), f32)
    zi16 = jnp.zeros((16,), i32)
    iota16 = lax.iota(i32, 16)
    blocks = MW // 16

    def _zz(i, carry):
        r = jnp.full((16,), i // blocks, i32)
        cc = iota16 + (i % blocks) * 16
        plsc.store_scatter(zbuf, [r, cc], z16)
        return carry

    lax.fori_loop(0, ZR * blocks, _zz, 0)

    def _zc(j, carry):
        pltpu.sync_copy(zbuf, m_sh.at[pl.ds(sid * RPT + j * ZR, ZR)])
        return carry

    lax.fori_loop(0, RPT // ZR, _zc, 0)
    plsc.subcore_barrier()

    def _chunk(j, carry):
        chunkid = j * NW + wid

        @pl.when(chunkid < NCHG)
        def _():
            cidxb[pl.ds(0, 16)] = jnp.full((16,), chunkid, i32)
            ci1 = pltpu.async_copy(src2_hbm.at[cidxb], srcb2, s1)
            ci2 = pltpu.async_copy(dst2_hbm.at[cidxb], dstb2, s2)
            ci1.wait()
            ci2.wait()

            def _sub(s, scarry):
                soff = s * SB

                def _ic(i, icarry):
                    col = iota16 + soff + i * 16
                    srcA[pl.ds(i * 16, 16)] = plsc.load_gather(
                        srcb2, [zi16, col])
                    dstA[pl.ds(i * 16, 16)] = plsc.load_gather(
                        dstb2, [zi16, col])
                    return icarry

                lax.fori_loop(0, SB // 16, _ic, 0)
                c1 = pltpu.async_copy(q_hbm.at[dstA], qrows, s1)
                c2 = pltpu.async_copy(k_hbm.at[srcA], krows, s2)
                c3 = pltpu.async_copy(v_hbm.at[srcA], vrows, s3)
                c1.wait()
                c2.wait()
                c3.wait()

                def _group(g, gcarry):
                    rows = iota16 + g * 16
                    acc = jnp.zeros((16,), f32)
                    for c in range(H):
                        cc = jnp.full((16,), c, i32)
                        acc = acc + (plsc.load_gather(qrows, [rows, cc])
                                     * plsc.load_gather(krows, [rows, cc]))
                    exv = jnp.exp(jnp.maximum(acc * _SCALE, 0.0))
                    exb[pl.ds(soff + g * 16, 16)] = exv
                    return gcarry

                lax.fori_loop(0, SB // 16, _group, 0)

                def _scale(e, ecarry):
                    a = plsc.load_gather(exb, [jnp.full((16,), soff + e, i32)])
                    for b in range(MW // 16):
                        vmsg[e, pl.ds(b * 16, 16)] = (
                            vrows[e, pl.ds(b * 16, 16)] * a)
                    return ecarry

                lax.fori_loop(0, SB, _scale, 0)
                pltpu.sync_copy(vmsg, m_sh.at[dstA], add=True)
                return scarry

            lax.fori_loop(0, CE // SB, _sub, 0)
            pltpu.sync_copy(exb, ex_hbm.at[wid, j, 0])

        return carry

    lax.fori_loop(0, JMAX_E, _chunk, 0)

    plsc.subcore_barrier()

    def _wb(j, carry):
        r0 = sid * RPT + j * ZR
        pltpu.sync_copy(m_sh.at[pl.ds(r0, ZR)], mp_hbm.at[cid, pl.ds(r0, ZR)])
        return carry

    lax.fori_loop(0, RPT // ZR, _wb, 0)


@functools.partial(
    pl.kernel,
    out_type=jax.ShapeDtypeStruct((E,), f32),
    mesh=_sc_mesh,
    scratch_types=[
        pltpu.VMEM((CN,), i32),
        pltpu.VMEM((CN,), f32),
        pltpu.VMEM((CN,), f32),
        pltpu.VMEM((N,), f32),
    ],
    compiler_params=pltpu.CompilerParams(needs_layout_passes=False),
)
def _norm_pass(ex_hbm, dst_hbm, denom_hbm, a_hbm, dstb, exb, ab, denom_t):
    cid = lax.axis_index("c")
    sid = lax.axis_index("s")
    wid = cid * NS + sid
    pltpu.sync_copy(denom_hbm, denom_t)

    def _chunk(j, carry):
        chunkid = j * NW + wid

        @pl.when(chunkid < NCHN)
        def _():
            base = pl.multiple_of(chunkid * CN, CN)
            pltpu.sync_copy(dst_hbm.at[pl.ds(base, CN)], dstb)
            pltpu.sync_copy(ex_hbm.at[pl.ds(base, CN)], exb)

            def _g(g, gcarry):
                dstv = dstb[pl.ds(g * 16, 16)]
                exv = exb[pl.ds(g * 16, 16)]
                d = plsc.load_gather(denom_t, [dstv])
                ab[pl.ds(g * 16, 16)] = exv / (d + _EPS)
                return gcarry

            lax.fori_loop(0, GN, _g, 0)
            pltpu.sync_copy(ab, a_hbm.at[pl.ds(base, CN)])

        return carry

    lax.fori_loop(0, JMAX_N, _chunk, 0)


def _unpack_mp(mp32):
    b = jax.lax.bitcast_convert_type(mp32, jnp.bfloat16)
    return b.reshape(2, NP, MW).astype(f32)


def kernel(x, edge_index, W_proj, b_proj, Wq1, Wk1, Wv1, Wq2, Wk2, Wv2, Ws, bs):
    src = edge_index[0]
    dst = edge_index[1]

    q1, k1, v1 = pl.pallas_call(
        _dense1_body,
        out_shape=[jax.ShapeDtypeStruct((N, VW), f32),
                   jax.ShapeDtypeStruct((N, VW), f32),
                   jax.ShapeDtypeStruct((N, VW), f32)],
    )(x, W_proj, b_proj.reshape(1, H), Wq1, Wk1, Wv1)

    src2 = src.reshape(NCHG, CE)
    dst2 = dst.reshape(NCHG, CE)
    ex1, mp1 = _edge_pass(q1, k1, v1, src2, dst2)
    ex1 = jnp.transpose(ex1[:, :, 0, :], (1, 0, 2)).reshape(-1)[:E]

    m1 = mp1[0, :N] + mp1[1, :N]
    h1 = _gelu(m1[:, :H] / (m1[:, H:H + 1] + _EPS))

    q2, k2, v2 = pl.pallas_call(
        _mid_body,
        out_shape=[jax.ShapeDtypeStruct((N, VW), f32),
                   jax.ShapeDtypeStruct((N, VW), f32),
                   jax.ShapeDtypeStruct((N, VW), f32)],
    )(h1, Wq2, Wk2, Wv2)

    a1 = ex1 / (jax.ops.segment_sum(ex1, dst, num_segments=N)[dst] + _EPS)

    ex2, mp2 = _edge_pass(q2, k2, v2, src2, dst2)
    ex2 = jnp.transpose(ex2[:, :, 0, :], (1, 0, 2)).reshape(-1)[:E]

    m2 = mp2[0, :N] + mp2[1, :N]
    h2 = m2[:, :H] / (m2[:, H:H + 1] + _EPS)

    signals = pl.pallas_call(
        _fin_body,
        out_shape=jax.ShapeDtypeStruct((N, 1), f32),
    )(h2, Ws, bs.reshape(1, 1))

    a2 = ex2 / (jax.ops.segment_sum(ex2, dst, num_segments=N)[dst] + _EPS)

    return (signals, a1[:, None], a2[:, None])
